# Initial kernel scaffold; baseline (speedup 1.0000x reference)
import jax
import jax.numpy as jnp
from jax.experimental import pallas as pl


def kernel(x_re, x_im):
    raise NotImplementedError("write your pallas kernel here")


# speedup vs baseline: 26.9332x; 26.9332x over previous
import math

import jax
import jax.numpy as jnp
import numpy as np
from jax.experimental import pallas as pl
from jax.experimental.pallas import tpu as pltpu

_N = 256
_ROWS = 32768
_BLK = 2048


def _dft_mats():
    k = np.arange(_N, dtype=np.float64)
    ang = -2.0 * math.pi * np.outer(k, k) / _N
    return (np.cos(ang).astype(np.float32), np.sin(ang).astype(np.float32))


def _fft_block_kernel(xr_ref, xi_ref, fr_ref, fi_ref, yr_ref, yi_ref):
    xr = xr_ref[...]
    xi = xi_ref[...]
    fr = fr_ref[...]
    fi = fi_ref[...]
    yr_ref[...] = jax.lax.dot(
        xr, fr, preferred_element_type=jnp.float32
    ) - jax.lax.dot(xi, fi, preferred_element_type=jnp.float32)
    yi_ref[...] = jax.lax.dot(
        xr, fi, preferred_element_type=jnp.float32
    ) + jax.lax.dot(xi, fr, preferred_element_type=jnp.float32)


def kernel(x_re, x_im):
    fr, fi = _dft_mats()
    fr = jnp.asarray(fr)
    fi = jnp.asarray(fi)
    grid = _ROWS // _BLK
    out = pl.pallas_call(
        _fft_block_kernel,
        grid=(grid,),
        in_specs=[
            pl.BlockSpec((_BLK, _N), lambda i: (i, 0)),
            pl.BlockSpec((_BLK, _N), lambda i: (i, 0)),
            pl.BlockSpec((_N, _N), lambda i: (0, 0)),
            pl.BlockSpec((_N, _N), lambda i: (0, 0)),
        ],
        out_specs=[
            pl.BlockSpec((_BLK, _N), lambda i: (i, 0)),
            pl.BlockSpec((_BLK, _N), lambda i: (i, 0)),
        ],
        out_shape=[
            jax.ShapeDtypeStruct((_ROWS, _N), jnp.float32),
            jax.ShapeDtypeStruct((_ROWS, _N), jnp.float32),
        ],
    )(x_re, x_im, fr, fi)
    return (out[0], out[1])
